# baseline (speedup 1.0000x reference)
import jax
import jax.numpy as jnp
from jax import lax
from jax.experimental import pallas as pl

_K = 24
_D = 12
_C = 25
_KP = 32
_L = 8192


def _body(x_ref, w_ref, cn_ref, out_ref):
    x = x_ref[...]
    t = lax.dot_general(
        w_ref[...], x,
        dimension_numbers=(((1,), (0,)), ((), ())),
        preferred_element_type=jnp.float32,
        precision=lax.Precision.HIGHEST,
    )
    t = t - cn_ref[...]
    kv = lax.broadcasted_iota(jnp.int32, (_KP, 1), 0)

    half = _KP // 2
    a = t[:half, :]
    b = t[half:, :]
    cmp = a >= b
    val = jnp.where(cmp, a, b)
    idx = jnp.where(cmp, kv[:half, :], kv[half:, :])
    half //= 2
    while half >= 1:
        a = val[:half, :]
        b = val[half:, :]
        ia = idx[:half, :]
        ib = idx[half:, :]
        cmp = (a > b) | ((a == b) & (ia < ib))
        val = jnp.where(cmp, a, b)
        idx = jnp.where(cmp, ia, ib)
        half //= 2

    lbl = idx + 1
    cls = lax.broadcasted_iota(jnp.int32, (_C, 1), 0)
    out_ref[...] = (cls == lbl).astype(jnp.float32)


def kernel(inputs, CTT):
    n = inputs.shape[0]
    grid = n // _L
    f32 = jnp.float32

    w = jnp.concatenate([2.0 * CTT, jnp.zeros((_KP - _K, _D), f32)], axis=0)
    cn = jnp.sum(CTT * CTT, axis=1)
    cnp = jnp.concatenate([cn, jnp.full((_KP - _K,), 1e30, f32)])[:, None]

    xt = inputs.T

    res = pl.pallas_call(
        _body,
        grid=(grid,),
        in_specs=[
            pl.BlockSpec((_D, _L), lambda i: (0, i)),
            pl.BlockSpec((_KP, _D), lambda i: (0, 0)),
            pl.BlockSpec((_KP, 1), lambda i: (0, 0)),
        ],
        out_specs=pl.BlockSpec((_C, _L), lambda i: (0, i)),
        out_shape=jax.ShapeDtypeStruct((_C, n), f32),
    )(xt, w, cnp)
    return res.T


# speedup vs baseline: 1.0015x; 1.0015x over previous
import jax
import jax.numpy as jnp
from jax import lax
from jax.experimental import pallas as pl

_K = 24
_D = 12
_C = 25
_S = 32
_L = 8192


def _body(x_ref, w_ref, cn_ref, out_ref):
    x = x_ref[...]
    t = lax.dot_general(
        w_ref[...], x,
        dimension_numbers=(((1,), (0,)), ((), ())),
        preferred_element_type=jnp.float32,
        precision=lax.Precision.HIGHEST,
    )
    t = t - cn_ref[...]

    m = jnp.maximum(t[:16, :], t[16:, :])
    m = jnp.maximum(m[:8, :], m[8:, :])
    m = jnp.maximum(m[:4, :], m[4:, :])
    m = jnp.maximum(m[:2, :], m[2:, :])
    m = jnp.maximum(m[:1, :], m[1:, :])

    out_ref[...] = (t[:_C, :] == m).astype(jnp.float32)


def kernel(inputs, CTT):
    n = inputs.shape[0]
    grid = n // _L
    f32 = jnp.float32

    w = jnp.concatenate(
        [jnp.zeros((1, _D), f32), 2.0 * CTT, jnp.zeros((_S - 1 - _K, _D), f32)],
        axis=0,
    )
    cn = jnp.sum(CTT * CTT, axis=1)
    big = jnp.full((1,), 1e30, f32)
    cnp = jnp.concatenate([big, cn, jnp.full((_S - 1 - _K,), 1e30, f32)])[:, None]

    xt = inputs.T

    res = pl.pallas_call(
        _body,
        grid=(grid,),
        in_specs=[
            pl.BlockSpec((_D, _L), lambda i: (0, i)),
            pl.BlockSpec((_S, _D), lambda i: (0, 0)),
            pl.BlockSpec((_S, 1), lambda i: (0, 0)),
        ],
        out_specs=pl.BlockSpec((_C, _L), lambda i: (0, i)),
        out_shape=jax.ShapeDtypeStruct((_C, n), f32),
    )(xt, w, cnp)
    return res.T


# speedup vs baseline: 1.2089x; 1.2071x over previous
import jax
import jax.numpy as jnp
from jax import lax
from jax.experimental import pallas as pl

_K = 24
_D = 12
_C = 25
_S = 32
_L = 16384


def _body(x_ref, w_ref, cn_ref, out_ref):
    x = x_ref[...]
    t = lax.dot_general(
        w_ref[...], x,
        dimension_numbers=(((1,), (0,)), ((), ())),
        preferred_element_type=jnp.float32,
        precision=lax.Precision.HIGHEST,
    )
    t = t - cn_ref[...]

    m = jnp.maximum(t[:16, :], t[16:, :])
    m = jnp.maximum(m[:8, :], m[8:, :])
    m = jnp.maximum(m[:4, :], m[4:, :])
    m = jnp.maximum(m[:2, :], m[2:, :])
    m = jnp.maximum(m[:1, :], m[1:, :])

    out_ref[...] = (t[:_C, :] == m).astype(jnp.float32)


def kernel(inputs, CTT):
    n = inputs.shape[0]
    grid = n // _L
    f32 = jnp.float32

    w = jnp.concatenate(
        [jnp.zeros((1, _D), f32), 2.0 * CTT, jnp.zeros((_S - 1 - _K, _D), f32)],
        axis=0,
    )
    cn = jnp.sum(CTT * CTT, axis=1)
    big = jnp.full((1,), 1e30, f32)
    cnp = jnp.concatenate([big, cn, jnp.full((_S - 1 - _K,), 1e30, f32)])[:, None]

    xt = inputs.T

    res = pl.pallas_call(
        _body,
        grid=(grid,),
        in_specs=[
            pl.BlockSpec((_D, _L), lambda i: (0, i)),
            pl.BlockSpec((_S, _D), lambda i: (0, 0)),
            pl.BlockSpec((_S, 1), lambda i: (0, 0)),
        ],
        out_specs=pl.BlockSpec((_C, _L), lambda i: (0, i)),
        out_shape=jax.ShapeDtypeStruct((_C, n), f32),
    )(xt, w, cnp)
    return res.T


# speedup vs baseline: 1.2748x; 1.0545x over previous
import jax
import jax.numpy as jnp
from jax import lax
from jax.experimental import pallas as pl

_K = 24
_D = 12
_C = 25
_S = 32
_L = 32768


def _body(x_ref, w_ref, cn_ref, out_ref):
    x = x_ref[...]
    t = lax.dot_general(
        w_ref[...], x,
        dimension_numbers=(((1,), (0,)), ((), ())),
        preferred_element_type=jnp.float32,
        precision=lax.Precision.HIGHEST,
    )
    t = t - cn_ref[...]

    m = jnp.maximum(t[:16, :], t[16:, :])
    m = jnp.maximum(m[:8, :], m[8:, :])
    m = jnp.maximum(m[:4, :], m[4:, :])
    m = jnp.maximum(m[:2, :], m[2:, :])
    m = jnp.maximum(m[:1, :], m[1:, :])

    out_ref[...] = (t[:_C, :] == m).astype(jnp.float32)


def kernel(inputs, CTT):
    n = inputs.shape[0]
    grid = n // _L
    f32 = jnp.float32

    w = jnp.concatenate(
        [jnp.zeros((1, _D), f32), 2.0 * CTT, jnp.zeros((_S - 1 - _K, _D), f32)],
        axis=0,
    )
    cn = jnp.sum(CTT * CTT, axis=1)
    big = jnp.full((1,), 1e30, f32)
    cnp = jnp.concatenate([big, cn, jnp.full((_S - 1 - _K,), 1e30, f32)])[:, None]

    xt = inputs.T

    res = pl.pallas_call(
        _body,
        grid=(grid,),
        in_specs=[
            pl.BlockSpec((_D, _L), lambda i: (0, i)),
            pl.BlockSpec((_S, _D), lambda i: (0, 0)),
            pl.BlockSpec((_S, 1), lambda i: (0, 0)),
        ],
        out_specs=pl.BlockSpec((_C, _L), lambda i: (0, i)),
        out_shape=jax.ShapeDtypeStruct((_C, n), f32),
    )(xt, w, cnp)
    return res.T
